# baseline (speedup 1.0000x reference)
import functools

import jax
import jax.numpy as jnp
from jax import lax
from jax.experimental import pallas as pl
from jax.experimental.pallas import tpu as pltpu

NUM_LAYERS = 3


def _dot(a, b):
    return lax.dot_general(a, b, (((1,), (0,)), ((), ())),
                           preferred_element_type=jnp.float32)


def _dot_t(a, b):
    return lax.dot_general(a, b, (((0,), (0,)), ((), ())),
                           preferred_element_type=jnp.float32)


def _ggnn_kernel(x_ref, adj_ref, wcat_ref, wih_ref, bih_ref, bhh_ref,
                 out_ref, *, num_layers, d):
    h = x_ref[0]
    A = adj_ref[0].astype(jnp.float32)
    b_ih = bih_ref[:, :]
    b_hh = bhh_ref[:, :]
    for l in range(num_layers):
        cat = _dot(h, wcat_ref[l])
        m = cat[:, 0:d]
        gh = cat[:, d:4 * d] + b_hh
        agg = _dot_t(A, m)
        gi = _dot(agg, wih_ref[:, :]) + b_ih
        r = jax.nn.sigmoid(gi[:, 0:d] + gh[:, 0:d])
        z = jax.nn.sigmoid(gi[:, d:2 * d] + gh[:, d:2 * d])
        n = jnp.tanh(gi[:, 2 * d:3 * d] + r * gh[:, 2 * d:3 * d])
        h = (1.0 - z) * n + z * h
    out_ref[0] = h


def kernel(x, adj, W, W_ih, W_hh, b_ih, b_hh):
    B, N, D = x.shape
    num_layers = W.shape[0]
    wcat = jnp.concatenate(
        [W, jnp.broadcast_to(W_hh.T[None], (num_layers, D, 3 * D))], axis=2)
    wih_t = W_ih.T
    b_ih2 = b_ih.reshape(1, 3 * D)
    b_hh2 = b_hh.reshape(1, 3 * D)
    return pl.pallas_call(
        functools.partial(_ggnn_kernel, num_layers=num_layers, d=D),
        grid=(B,),
        in_specs=[
            pl.BlockSpec((1, N, D), lambda b: (b, 0, 0)),
            pl.BlockSpec((1, N, N), lambda b: (b, 0, 0)),
            pl.BlockSpec((num_layers, D, 4 * D), lambda b: (0, 0, 0)),
            pl.BlockSpec((D, 3 * D), lambda b: (0, 0)),
            pl.BlockSpec((1, 3 * D), lambda b: (0, 0)),
            pl.BlockSpec((1, 3 * D), lambda b: (0, 0)),
        ],
        out_specs=pl.BlockSpec((1, N, D), lambda b: (b, 0, 0)),
        out_shape=jax.ShapeDtypeStruct((B, N, D), jnp.float32),
        compiler_params=pltpu.CompilerParams(
            dimension_semantics=("parallel",)),
    )(x, adj, wcat, wih_t, b_ih2, b_hh2)


# speedup vs baseline: 1.1493x; 1.1493x over previous
import functools

import jax
import jax.numpy as jnp
from jax import lax
from jax.experimental import pallas as pl
from jax.experimental.pallas import tpu as pltpu

NUM_LAYERS = 3
GRAPHS_PER_STEP = 4


def _dot(a, b):
    return lax.dot_general(a, b, (((1,), (0,)), ((), ())),
                           preferred_element_type=jnp.float32)


def _dot_t(a, b):
    return lax.dot_general(a, b, (((0,), (0,)), ((), ())),
                           preferred_element_type=jnp.float32)


def _ggnn_kernel(x_ref, adj_ref, wcat_ref, wih_ref, bih_ref, bhh_ref,
                 out_ref, *, num_layers, d, n, g):
    h = x_ref[:, :, :].reshape(g * n, d)
    A = adj_ref[:, :, :].astype(jnp.float32)
    b_ih = bih_ref[:, :]
    b_hh = bhh_ref[:, :]
    for l in range(num_layers):
        cat = _dot(h, wcat_ref[l])
        m = cat[:, 0:d]
        gh = cat[:, d:4 * d] + b_hh
        aggs = [_dot_t(A[i], m[i * n:(i + 1) * n, :]) for i in range(g)]
        agg = jnp.concatenate(aggs, axis=0)
        gi = _dot(agg, wih_ref[:, :]) + b_ih
        r = jax.nn.sigmoid(gi[:, 0:d] + gh[:, 0:d])
        z = jax.nn.sigmoid(gi[:, d:2 * d] + gh[:, d:2 * d])
        nn = jnp.tanh(gi[:, 2 * d:3 * d] + r * gh[:, 2 * d:3 * d])
        h = (1.0 - z) * nn + z * h
    out_ref[:, :, :] = h.reshape(g, n, d)


def kernel(x, adj, W, W_ih, W_hh, b_ih, b_hh):
    B, N, D = x.shape
    num_layers = W.shape[0]
    g = min(GRAPHS_PER_STEP, B)
    wcat = jnp.concatenate(
        [W, jnp.broadcast_to(W_hh.T[None], (num_layers, D, 3 * D))], axis=2)
    wih_t = W_ih.T
    b_ih2 = b_ih.reshape(1, 3 * D)
    b_hh2 = b_hh.reshape(1, 3 * D)
    return pl.pallas_call(
        functools.partial(_ggnn_kernel, num_layers=num_layers, d=D, n=N, g=g),
        grid=(B // g,),
        in_specs=[
            pl.BlockSpec((g, N, D), lambda b: (b, 0, 0)),
            pl.BlockSpec((g, N, N), lambda b: (b, 0, 0)),
            pl.BlockSpec((num_layers, D, 4 * D), lambda b: (0, 0, 0)),
            pl.BlockSpec((D, 3 * D), lambda b: (0, 0)),
            pl.BlockSpec((1, 3 * D), lambda b: (0, 0)),
            pl.BlockSpec((1, 3 * D), lambda b: (0, 0)),
        ],
        out_specs=pl.BlockSpec((g, N, D), lambda b: (b, 0, 0)),
        out_shape=jax.ShapeDtypeStruct((B, N, D), jnp.float32),
    )(x, adj, wcat, wih_t, b_ih2, b_hh2)


# speedup vs baseline: 1.1951x; 1.0399x over previous
import functools

import jax
import jax.numpy as jnp
from jax import lax
from jax.experimental import pallas as pl
from jax.experimental.pallas import tpu as pltpu

NUM_LAYERS = 3
GRAPHS_PER_STEP = 2


def _dot(a, b):
    return lax.dot_general(a, b, (((1,), (0,)), ((), ())),
                           preferred_element_type=jnp.float32)


def _dot_t(a, b):
    return lax.dot_general(a, b, (((0,), (0,)), ((), ())),
                           preferred_element_type=jnp.float32)


def _ggnn_kernel(x_ref, adj_ref, wcat_ref, wih_ref, bih_ref, bhh_ref,
                 out_ref, *, num_layers, d, n, g):
    h = x_ref[:, :, :].reshape(g * n, d)
    A = adj_ref[:, :, :].astype(jnp.float32)
    b_ih = bih_ref[:, :]
    b_hh = bhh_ref[:, :]
    for l in range(num_layers):
        cat = _dot(h, wcat_ref[l])
        m = cat[:, 0:d]
        gh = cat[:, d:4 * d] + b_hh
        aggs = [_dot_t(A[i], m[i * n:(i + 1) * n, :]) for i in range(g)]
        agg = jnp.concatenate(aggs, axis=0)
        gi = _dot(agg, wih_ref[:, :]) + b_ih
        r = jax.nn.sigmoid(gi[:, 0:d] + gh[:, 0:d])
        z = jax.nn.sigmoid(gi[:, d:2 * d] + gh[:, d:2 * d])
        nn = jnp.tanh(gi[:, 2 * d:3 * d] + r * gh[:, 2 * d:3 * d])
        h = (1.0 - z) * nn + z * h
    out_ref[:, :, :] = h.reshape(g, n, d)


def kernel(x, adj, W, W_ih, W_hh, b_ih, b_hh):
    B, N, D = x.shape
    num_layers = W.shape[0]
    g = min(GRAPHS_PER_STEP, B)
    wcat = jnp.concatenate(
        [W, jnp.broadcast_to(W_hh.T[None], (num_layers, D, 3 * D))], axis=2)
    wih_t = W_ih.T
    b_ih2 = b_ih.reshape(1, 3 * D)
    b_hh2 = b_hh.reshape(1, 3 * D)
    return pl.pallas_call(
        functools.partial(_ggnn_kernel, num_layers=num_layers, d=D, n=N, g=g),
        grid=(B // g,),
        in_specs=[
            pl.BlockSpec((g, N, D), lambda b: (b, 0, 0)),
            pl.BlockSpec((g, N, N), lambda b: (b, 0, 0)),
            pl.BlockSpec((num_layers, D, 4 * D), lambda b: (0, 0, 0)),
            pl.BlockSpec((D, 3 * D), lambda b: (0, 0)),
            pl.BlockSpec((1, 3 * D), lambda b: (0, 0)),
            pl.BlockSpec((1, 3 * D), lambda b: (0, 0)),
        ],
        out_specs=pl.BlockSpec((g, N, D), lambda b: (b, 0, 0)),
        out_shape=jax.ShapeDtypeStruct((B, N, D), jnp.float32),
    )(x, adj, wcat, wih_t, b_ih2, b_hh2)


# speedup vs baseline: 1.2111x; 1.0134x over previous
import functools

import jax
import jax.numpy as jnp
from jax import lax
from jax.experimental import pallas as pl

NUM_LAYERS = 3
GRAPHS_PER_STEP = 2


def _dot(a, b):
    return lax.dot_general(a, b, (((1,), (0,)), ((), ())),
                           preferred_element_type=jnp.float32)


def _dot_t(a, b):
    return lax.dot_general(a, b, (((0,), (0,)), ((), ())),
                           preferred_element_type=jnp.float32)


def _ggnn_kernel(x_ref, adj_ref, wcat_ref, wih_ref, bih_ref, bhh_ref,
                 out_ref, *, num_layers, d, n, g):
    bf = jnp.bfloat16
    h = x_ref[:, :, :].reshape(g * n, d)
    A = adj_ref[:, :, :].astype(jnp.float32).astype(bf)
    b_ih = bih_ref[:, :]
    b_hh = bhh_ref[:, :]
    wih = wih_ref[:, :]
    for l in range(num_layers):
        cat = _dot(h.astype(bf), wcat_ref[l])
        m = cat[:, 0:d].astype(bf)
        gh = cat[:, d:4 * d] + b_hh
        aggs = [_dot_t(A[i], m[i * n:(i + 1) * n, :]) for i in range(g)]
        agg = jnp.concatenate(aggs, axis=0)
        gi = _dot(agg.astype(bf), wih) + b_ih
        r = jax.nn.sigmoid(gi[:, 0:d] + gh[:, 0:d])
        z = jax.nn.sigmoid(gi[:, d:2 * d] + gh[:, d:2 * d])
        nn = jnp.tanh(gi[:, 2 * d:3 * d] + r * gh[:, 2 * d:3 * d])
        h = (1.0 - z) * nn + z * h
    out_ref[:, :, :] = h.reshape(g, n, d)


def kernel(x, adj, W, W_ih, W_hh, b_ih, b_hh):
    B, N, D = x.shape
    num_layers = W.shape[0]
    g = min(GRAPHS_PER_STEP, B)
    wcat = jnp.concatenate(
        [W, jnp.broadcast_to(W_hh.T[None], (num_layers, D, 3 * D))],
        axis=2).astype(jnp.bfloat16)
    wih_t = W_ih.T.astype(jnp.bfloat16)
    b_ih2 = b_ih.reshape(1, 3 * D)
    b_hh2 = b_hh.reshape(1, 3 * D)
    return pl.pallas_call(
        functools.partial(_ggnn_kernel, num_layers=num_layers, d=D, n=N, g=g),
        grid=(B // g,),
        in_specs=[
            pl.BlockSpec((g, N, D), lambda b: (b, 0, 0)),
            pl.BlockSpec((g, N, N), lambda b: (b, 0, 0)),
            pl.BlockSpec((num_layers, D, 4 * D), lambda b: (0, 0, 0)),
            pl.BlockSpec((D, 3 * D), lambda b: (0, 0)),
            pl.BlockSpec((1, 3 * D), lambda b: (0, 0)),
            pl.BlockSpec((1, 3 * D), lambda b: (0, 0)),
        ],
        out_specs=pl.BlockSpec((g, N, D), lambda b: (b, 0, 0)),
        out_shape=jax.ShapeDtypeStruct((B, N, D), jnp.float32),
    )(x, adj, wcat, wih_t, b_ih2, b_hh2)
